# baseline (speedup 1.0000x reference)
import functools

import jax
import jax.numpy as jnp
from jax import lax
from jax.experimental import pallas as pl
from jax.experimental.pallas import tpu as pltpu
from jax.experimental.pallas import tpu_sc as plsc

Q = 1024
D = 64
KB = 4096
_PAD_VAL = 1e18
_I32_MAX = 2**31 - 1


def _argmin_body(xT_ref, w_ref, iota_ref, idx_ref, best_val, best_idx,
                 *, n_rows, n_blocks):
    k = pl.program_id(0)

    @pl.when(k == 0)
    def _init():
        best_val[...] = jnp.full_like(best_val, float("inf"))
        best_idx[...] = jnp.full_like(best_idx, float("inf"))

    xT = xT_ref[...]
    w = w_ref[...]
    roww = k * KB + lax.broadcasted_iota(jnp.int32, (KB, D), 0)
    w = jnp.where(roww < n_rows, w, _PAD_VAL)
    w2 = w + w
    qb2 = lax.dot_general(w2, xT, (((1,), (0,)), ((), ())),
                          preferred_element_type=jnp.float32)
    q2 = jnp.sum(xT * xT, axis=0, keepdims=True)
    b2 = jnp.sum(w * w, axis=1, keepdims=True)
    s = (q2 + b2) - qb2
    cmin = jnp.min(s, axis=0, keepdims=True)
    cidx = jnp.min(jnp.where(s == cmin, iota_ref[...], float("inf")),
                   axis=0, keepdims=True)
    improved = cmin < best_val[...]
    best_idx[...] = jnp.where(improved, cidx + jnp.float32(k * KB),
                              best_idx[...])
    best_val[...] = jnp.where(improved, cmin, best_val[...])

    @pl.when(k == n_blocks - 1)
    def _emit():
        idx_ref[...] = best_idx[...].astype(jnp.int32)


def _argmin_call(inputs_t, buffer):
    n_rows = buffer.shape[0]
    n_blocks = pl.cdiv(n_rows, KB)
    return pl.pallas_call(
        functools.partial(_argmin_body, n_rows=n_rows, n_blocks=n_blocks),
        grid=(n_blocks,),
        in_specs=[
            pl.BlockSpec((D, Q), lambda k: (0, 0)),
            pl.BlockSpec((KB, D), lambda k: (k, 0)),
            pl.BlockSpec((KB, Q), lambda k: (0, 0)),
        ],
        out_specs=pl.BlockSpec((1, Q), lambda k: (0, 0)),
        out_shape=jax.ShapeDtypeStruct((1, Q), jnp.int32),
        scratch_shapes=[
            pltpu.VMEM((1, Q), jnp.float32),
            pltpu.VMEM((1, Q), jnp.float32),
        ],
        compiler_params=pltpu.CompilerParams(
            dimension_semantics=("arbitrary",),
        ),
    )(inputs_t, buffer,
      lax.broadcasted_iota(jnp.float32, (KB, Q), 0))


def _sc_gather(buffer, idx):
    info = plsc.get_sparse_core_info()
    nc, ns = info.num_cores, info.num_subcores
    nw = nc * ns
    bpw = Q // nw
    mesh = plsc.VectorSubcoreMesh(core_axis_name="c", subcore_axis_name="s")

    @functools.partial(
        pl.kernel,
        mesh=mesh,
        out_type=jax.ShapeDtypeStruct((Q, D), jnp.float32),
        scratch_types=[
            pltpu.VMEM((bpw,), jnp.int32),
            pltpu.VMEM((bpw, D), jnp.float32),
            pltpu.SemaphoreType.DMA,
        ],
        compiler_params=pltpu.CompilerParams(use_tc_tiling_on_sc=False),
    )
    def gather_kernel(table_hbm, idx_hbm, out_hbm, idx_v, rows_v, sem):
        wid = lax.axis_index("s") * nc + lax.axis_index("c")
        base = wid * bpw
        pltpu.sync_copy(idx_hbm.at[pl.ds(base, bpw)], idx_v)
        pltpu.async_copy(table_hbm.at[idx_v], rows_v, sem).wait()
        pltpu.sync_copy(rows_v, out_hbm.at[pl.ds(base, bpw)])

    return gather_kernel(buffer, idx)


def kernel(inputs, buffer):
    idx = _argmin_call(inputs.T, buffer)[0]
    return _sc_gather(buffer, idx)


# speedup vs baseline: 1.0331x; 1.0331x over previous
import functools

import jax
import jax.numpy as jnp
from jax import lax
from jax.experimental import pallas as pl
from jax.experimental.pallas import tpu as pltpu
from jax.experimental.pallas import tpu_sc as plsc

Q = 1024
D = 64
KB = 4096
_PAD_VAL = 1e18
_I32_MAX = 2**31 - 1


def _argmin_body(xT_ref, w_ref, idx_ref, best_val, best_idx,
                 *, n_rows, n_blocks):
    k = pl.program_id(0)

    @pl.when(k == 0)
    def _init():
        best_val[...] = jnp.full_like(best_val, float("inf"))
        best_idx[...] = jnp.full_like(best_idx, _I32_MAX)

    xT = xT_ref[...]
    w = w_ref[...]
    roww = k * KB + lax.broadcasted_iota(jnp.int32, (KB, D), 0)
    w = jnp.where(roww < n_rows, w, _PAD_VAL)
    w2 = w + w
    qb2 = lax.dot_general(w2, xT, (((1,), (0,)), ((), ())),
                          preferred_element_type=jnp.float32)
    q2 = jnp.sum(xT * xT, axis=0, keepdims=True)
    b2 = jnp.sum(w * w, axis=1, keepdims=True)
    s = (q2 + b2) - qb2
    cmin = jnp.min(s, axis=0, keepdims=True)
    loc = lax.broadcasted_iota(jnp.int32, (KB, Q), 0)
    cidx = jnp.min(jnp.where(s == cmin, loc, _I32_MAX), axis=0,
                   keepdims=True)
    improved = cmin < best_val[...]
    best_idx[...] = jnp.where(improved, cidx + k * KB, best_idx[...])
    best_val[...] = jnp.where(improved, cmin, best_val[...])

    @pl.when(k == n_blocks - 1)
    def _emit():
        idx_ref[...] = best_idx[...]


def _argmin_call(inputs_t, buffer):
    n_rows = buffer.shape[0]
    n_blocks = pl.cdiv(n_rows, KB)
    return pl.pallas_call(
        functools.partial(_argmin_body, n_rows=n_rows, n_blocks=n_blocks),
        grid=(n_blocks,),
        in_specs=[
            pl.BlockSpec((D, Q), lambda k: (0, 0)),
            pl.BlockSpec((KB, D), lambda k: (k, 0)),
        ],
        out_specs=pl.BlockSpec((1, Q), lambda k: (0, 0)),
        out_shape=jax.ShapeDtypeStruct((1, Q), jnp.int32),
        scratch_shapes=[
            pltpu.VMEM((1, Q), jnp.float32),
            pltpu.VMEM((1, Q), jnp.int32),
        ],
        compiler_params=pltpu.CompilerParams(
            dimension_semantics=("arbitrary",),
        ),
    )(inputs_t, buffer)


def _sc_gather(buffer, idx):
    info = plsc.get_sparse_core_info()
    nc, ns = info.num_cores, info.num_subcores
    nw = nc * ns
    bpw = Q // nw
    mesh = plsc.VectorSubcoreMesh(core_axis_name="c", subcore_axis_name="s")

    @functools.partial(
        pl.kernel,
        mesh=mesh,
        out_type=jax.ShapeDtypeStruct((Q, D), jnp.float32),
        scratch_types=[
            pltpu.VMEM((bpw,), jnp.int32),
            pltpu.VMEM((bpw, D), jnp.float32),
            pltpu.SemaphoreType.DMA,
        ],
        compiler_params=pltpu.CompilerParams(use_tc_tiling_on_sc=False),
    )
    def gather_kernel(table_hbm, idx_hbm, out_hbm, idx_v, rows_v, sem):
        wid = lax.axis_index("s") * nc + lax.axis_index("c")
        base = wid * bpw
        pltpu.sync_copy(idx_hbm.at[pl.ds(base, bpw)], idx_v)
        pltpu.async_copy(table_hbm.at[idx_v], rows_v, sem).wait()
        pltpu.sync_copy(rows_v, out_hbm.at[pl.ds(base, bpw)])

    return gather_kernel(buffer, idx)


def kernel(inputs, buffer):
    idx = _argmin_call(inputs.T, buffer)[0]
    return _sc_gather(buffer, idx)
